# Initial kernel scaffold; baseline (speedup 1.0000x reference)
import jax
import jax.numpy as jnp
from jax.experimental import pallas as pl


def kernel(z, edge_index, W1, b1, W2, b2, W3, b3, W4, b4):
    raise NotImplementedError("write your pallas kernel here")


# speedup vs baseline: 7.0662x; 7.0662x over previous
import functools

import jax
import jax.numpy as jnp
from jax import lax
from jax.experimental import pallas as pl
from jax.experimental.pallas import tpu as pltpu
from jax.experimental.pallas import tpu_sc as plsc

N = 10000
E = 320000
NP = 10240
L = 128
NC = 2
NS = 16
NW = NC * NS
CH = 79
EP = NW * CH * L
NSTRIPE = NP // NS


def _mesh():
    return plsc.VectorSubcoreMesh(
        core_axis_name="c", subcore_axis_name="s", num_cores=NC, num_subcores=NS
    )


def _sc_degrees(srcr, dstr, ones_l, zeros_np):
    def body(src_hbm, dst_hbm, ones_hbm, z_hbm, dsrc_out, ddst_out,
             sidx, didx, ones_v, dsrc_sh, ddst_sh):
        c = lax.axis_index("c")
        s = lax.axis_index("s")
        w = s * NC + c
        stripe = pl.ds(s * NSTRIPE, NSTRIPE)
        pltpu.sync_copy(z_hbm.at[stripe], dsrc_sh.at[stripe])
        pltpu.sync_copy(z_hbm.at[stripe], ddst_sh.at[stripe])
        pltpu.sync_copy(ones_hbm, ones_v)
        pltpu.sync_copy(src_hbm.at[w], sidx)
        pltpu.sync_copy(dst_hbm.at[w], didx)
        plsc.subcore_barrier()

        def chunk(j, carry):
            pltpu.sync_copy(ones_v, dsrc_sh.at[sidx.at[j]], add=True)
            pltpu.sync_copy(ones_v, ddst_sh.at[didx.at[j]], add=True)
            return carry

        lax.fori_loop(0, CH, chunk, 0)
        plsc.subcore_barrier()
        pltpu.sync_copy(dsrc_sh.at[stripe], dsrc_out.at[c, stripe])
        pltpu.sync_copy(ddst_sh.at[stripe], ddst_out.at[c, stripe])

    f = pl.kernel(
        body,
        out_type=(
            jax.ShapeDtypeStruct((NC, NP), jnp.float32),
            jax.ShapeDtypeStruct((NC, NP), jnp.float32),
        ),
        mesh=_mesh(),
        scratch_types=[
            pltpu.VMEM((CH, L), jnp.int32),
            pltpu.VMEM((CH, L), jnp.int32),
            pltpu.VMEM((L,), jnp.float32),
            pltpu.VMEM_SHARED((NP,), jnp.float32),
            pltpu.VMEM_SHARED((NP,), jnp.float32),
        ],
    )
    return f(srcr, dstr, ones_l, zeros_np)


def _sc_aggregate(h, srcr, dstr, zeros_nd, d):
    def body(h_hbm, src_hbm, dst_hbm, z_hbm, agg_out,
             sidx, didx, rows, agg_sh, sem):
        c = lax.axis_index("c")
        s = lax.axis_index("s")
        w = s * NC + c
        stripe = pl.ds(s * NSTRIPE, NSTRIPE)
        pltpu.sync_copy(z_hbm.at[stripe], agg_sh.at[stripe])
        pltpu.sync_copy(src_hbm.at[w], sidx)
        pltpu.sync_copy(dst_hbm.at[w], didx)
        plsc.subcore_barrier()

        def chunk(j, carry):
            pltpu.async_copy(h_hbm.at[sidx.at[j]], rows, sem).wait()
            pltpu.sync_copy(rows, agg_sh.at[didx.at[j]], add=True)
            return carry

        lax.fori_loop(0, CH, chunk, 0)
        plsc.subcore_barrier()
        pltpu.sync_copy(agg_sh.at[stripe], agg_out.at[c, stripe])

    f = pl.kernel(
        body,
        out_type=jax.ShapeDtypeStruct((NC, NP, d), jnp.float32),
        mesh=_mesh(),
        compiler_params=pltpu.CompilerParams(use_tc_tiling_on_sc=False),
        scratch_types=[
            pltpu.VMEM((CH, L), jnp.int32),
            pltpu.VMEM((CH, L), jnp.int32),
            pltpu.VMEM((L, d), jnp.float32),
            pltpu.VMEM_SHARED((NP, d), jnp.float32),
            pltpu.SemaphoreType.DMA,
        ],
    )
    return f(h, srcr, dstr, zeros_nd)


def _tc_norms_h0(degs, degd, z_pad):
    def body(ds_ref, dd_ref, z_ref, ns_ref, nd_ref, h0_ref):
        dsum_s = ds_ref[0] + ds_ref[1]
        dsum_d = dd_ref[0] + dd_ref[1]
        ns = jnp.where(dsum_s > 0, lax.rsqrt(dsum_s), 0.0)
        nd = jnp.where(dsum_d > 0, lax.rsqrt(dsum_d), 0.0)
        ns_ref[...] = ns
        nd_ref[...] = nd
        h0_ref[...] = z_ref[...] * ns

    return pl.pallas_call(
        body,
        out_shape=(
            jax.ShapeDtypeStruct((NP, 1), jnp.float32),
            jax.ShapeDtypeStruct((NP, 1), jnp.float32),
            jax.ShapeDtypeStruct((NP, z_pad.shape[1]), jnp.float32),
        ),
    )(degs, degd, z_pad)


def _tc_layer(agg_part, W, b, norm_dst, norm_src, scale_src, d_out):
    def body(a_ref, w_ref, b_ref, nd_ref, ns_ref, o_ref):
        agg = a_ref[0] + a_ref[1]
        r = jnp.dot(agg, w_ref[...], preferred_element_type=jnp.float32)
        r = r * nd_ref[...] + b_ref[...]
        r = jnp.maximum(r, 0.0)
        if scale_src:
            r = r * ns_ref[...]
        o_ref[...] = r

    return pl.pallas_call(
        body,
        out_shape=jax.ShapeDtypeStruct((NP, d_out), jnp.float32),
    )(agg_part, W, b.reshape(1, -1), norm_dst, norm_src)


def kernel(z, edge_index, W1, b1, W2, b2, W3, b3, W4, b4):
    src = edge_index[0]
    dst = edge_index[1]
    pad = EP - E
    srcr = jnp.concatenate([src, jnp.full((pad,), N, jnp.int32)]).reshape(NW, CH, L)
    dstr = jnp.concatenate([dst, jnp.full((pad,), N, jnp.int32)]).reshape(NW, CH, L)
    z_pad = jnp.zeros((NP, z.shape[1]), jnp.float32).at[:N].set(z)

    ones_l = jnp.ones((L,), jnp.float32)
    zeros_np = jnp.zeros((NP,), jnp.float32)

    degs, degd = _sc_degrees(srcr, dstr, ones_l, zeros_np)
    ns, nd, h0 = _tc_norms_h0(degs[:, :, None], degd[:, :, None], z_pad)

    agg = _sc_aggregate(h0, srcr, dstr, jnp.zeros((NP, 32), jnp.float32), 32)
    h1 = _tc_layer(agg, W1, b1, nd, ns, True, 32)
    agg = _sc_aggregate(h1, srcr, dstr, jnp.zeros((NP, 32), jnp.float32), 32)
    h2 = _tc_layer(agg, W2, b2, nd, ns, True, 64)
    agg = _sc_aggregate(h2, srcr, dstr, jnp.zeros((NP, 64), jnp.float32), 64)
    h3 = _tc_layer(agg, W3, b3, nd, ns, True, 128)
    agg = _sc_aggregate(h3, srcr, dstr, jnp.zeros((NP, 128), jnp.float32), 128)
    x4 = _tc_layer(agg, W4, b4, nd, ns, False, 128)
    return x4[:N]
